# baseline (speedup 1.0000x reference)
import functools

import jax
import jax.numpy as jnp
from jax import lax
from jax.experimental import pallas as pl
from jax.experimental.pallas import tpu as pltpu
from jax.experimental.pallas import tpu_sc as plsc

B = 4
D = 128
H = 224
W = 224
NC = 2
NS = 16
NW = NC * NS
H_PER_W = H // NW
LANES = 16
DV = D // LANES
FAST_C = 0


def _pos_embed_sc(row_embed, col_embed):
    mesh = plsc.VectorSubcoreMesh(core_axis_name="c", subcore_axis_name="s")

    @functools.partial(
        pl.kernel,
        out_type=jax.ShapeDtypeStruct((B, H, W, D), jnp.float32),
        mesh=mesh,
        compiler_params=pltpu.CompilerParams(needs_layout_passes=False),
        scratch_types=[
            pltpu.VMEM((W, D), jnp.float32),
            pltpu.VMEM((H, D), jnp.float32),
            pltpu.VMEM((2, W, D), jnp.float32),
            pltpu.SemaphoreType.DMA((2,)),
        ],
    )
    def k(row_hbm, col_hbm, out_hbm, cols_v, rows_v, slab_v, sems):
        cid = lax.axis_index("c")
        sid = lax.axis_index("s")
        wid = sid * NC + cid
        h0 = wid * H_PER_W

        pltpu.sync_copy(col_hbm.at[pl.ds(0, W)], cols_v)
        pltpu.sync_copy(row_hbm.at[pl.ds(0, H)], rows_v)

        def drain(p, n):
            for _ in range(n):
                pltpu.make_async_copy(
                    out_hbm.at[0, 0], slab_v.at[0], sems.at[p]
                ).wait()

        def run(seq):
            n = len(seq)
            for s, (h, bs) in enumerate(seq):
                p = s % 2
                if s >= 2:
                    drain(p, len(seq[s - 2][1]))
                rvecs = [rows_v[h, pl.ds(i * LANES, LANES)]
                         for i in range(DV)]

                @plsc.parallel_loop(0, W, unroll=4)
                def body(w):
                    for i in range(DV):
                        sl = pl.ds(i * LANES, LANES)
                        slab_v[p, w, sl] = cols_v[w, sl] + rvecs[i]

                for b in bs:
                    pltpu.async_copy(
                        slab_v.at[p], out_hbm.at[b, h], sems.at[p]
                    )
            drain(n % 2, len(seq[n - 2][1]))
            drain((n + 1) % 2, len(seq[n - 1][1]))

        partner_h0 = (sid * NC + (1 - FAST_C)) * H_PER_W

        @pl.when(cid == FAST_C)
        def _():
            seq = [(partner_h0, (0, 1))]
            seq += [(h0 + i, (0, 1, 2, 3)) for i in range(H_PER_W)]
            run(seq)

        @pl.when(cid != FAST_C)
        def _():
            seq = [(h0, (2, 3))]
            seq += [(h0 + i, (0, 1, 2, 3)) for i in range(1, H_PER_W)]
            run(seq)

    return k(row_embed, col_embed)


def kernel(x, row_embed, col_embed):
    del x
    y = _pos_embed_sc(row_embed, col_embed)
    return jnp.transpose(y, (0, 3, 1, 2))


# speedup vs baseline: 1.1746x; 1.1746x over previous
import functools

import jax
import jax.numpy as jnp
from jax import lax
from jax.experimental import pallas as pl
from jax.experimental.pallas import tpu as pltpu
from jax.experimental.pallas import tpu_sc as plsc

B = 4
D = 128
H = 224
W = 224
NC = 2
NS = 16
NW = NC * NS
H_PER_W = H // NW
LANES = 16
DV = D // LANES


def _pos_embed_sc(row_embed, col_embed):
    mesh = plsc.VectorSubcoreMesh(core_axis_name="c", subcore_axis_name="s")

    @functools.partial(
        pl.kernel,
        out_type=jax.ShapeDtypeStruct((B, H, W, D), jnp.float32),
        mesh=mesh,
        compiler_params=pltpu.CompilerParams(needs_layout_passes=False),
        scratch_types=[
            pltpu.VMEM((W, D), jnp.float32),
            pltpu.VMEM((H, D), jnp.float32),
            pltpu.VMEM((2, W, D), jnp.float32),
            pltpu.SemaphoreType.DMA((2,)),
        ],
    )
    def k(row_hbm, col_hbm, out_hbm, cols_v, rows_v, slab_v, sems):
        wid = lax.axis_index("s") * NC + lax.axis_index("c")
        h0 = wid * H_PER_W

        c1 = pltpu.async_copy(col_hbm.at[pl.ds(0, W)], cols_v, sems.at[0])
        c2 = pltpu.async_copy(row_hbm.at[pl.ds(0, H)], rows_v, sems.at[0])
        c1.wait()
        c2.wait()

        def drain(p):
            for _ in range(B):
                pltpu.make_async_copy(
                    out_hbm.at[0, 0], slab_v.at[0], sems.at[p]
                ).wait()

        def loop_body(hh, carry):
            p = lax.rem(hh, 2)

            @pl.when(hh >= 2)
            def _():
                drain(p)

            rvecs = [rows_v[h0 + hh, pl.ds(i * LANES, LANES)]
                     for i in range(DV)]

            @plsc.parallel_loop(0, W, unroll=4)
            def body(w):
                for i in range(DV):
                    sl = pl.ds(i * LANES, LANES)
                    slab_v[p, w, sl] = cols_v[w, sl] + rvecs[i]

            for b in range(B):
                pltpu.async_copy(
                    slab_v.at[p], out_hbm.at[b, h0 + hh], sems.at[p]
                )
            return carry

        lax.fori_loop(0, H_PER_W, loop_body, 0)
        drain(jnp.int32((H_PER_W - 2) % 2))
        drain(jnp.int32((H_PER_W - 1) % 2))

    return k(row_embed, col_embed)


def kernel(x, row_embed, col_embed):
    del x
    y = _pos_embed_sc(row_embed, col_embed)
    return jnp.transpose(y, (0, 3, 1, 2))
